# baseline (speedup 1.0000x reference)
import functools

import jax
import jax.numpy as jnp
from jax import lax
from jax.experimental import pallas as pl
from jax.experimental.pallas import tpu as pltpu
from jax.experimental.pallas import tpu_sc as plsc

BATCH = 16384
FACTORS = 64
MLP_FACTOR = 32
NC, NS = 2, 16
NW = NC * NS
BPW = BATCH // NW


C = 256
NCHUNK = BPW // C
NBUF = 3


def _sc_gather(uidx_g, iidx_g, uidx_m, iidx_m, gu_tab, gi_tab, mu_tab, mi_tab):
    mesh = plsc.VectorSubcoreMesh(core_axis_name="c", subcore_axis_name="s")
    out_type = tuple(
        jax.ShapeDtypeStruct((BATCH, 128), jnp.float32) for _ in range(4))

    @functools.partial(
        pl.kernel,
        mesh=mesh,
        out_type=out_type,
        scratch_types=(
            [pltpu.VMEM((BPW,), jnp.int32)] * 4
            + [pltpu.VMEM((C, 128), jnp.float32)] * NBUF
            + [pltpu.SemaphoreType.DMA] * (2 * NBUF)
        ),
    )
    def k(ug_hbm, ig_hbm, um_hbm, im_hbm, gu_hbm, gi_hbm, mu_hbm, mi_hbm,
          ogu_hbm, ogi_hbm, omu_hbm, omi_hbm,
          ugx_v, igx_v, umx_v, imx_v, *bufs_and_sems):
        bufs = bufs_and_sems[:NBUF]
        gsems = bufs_and_sems[NBUF:2 * NBUF]
        osems = bufs_and_sems[2 * NBUF:]
        wid = lax.axis_index("s") * NC + lax.axis_index("c")
        base = wid * BPW
        pltpu.sync_copy(ug_hbm.at[pl.ds(base, BPW)], ugx_v)
        pltpu.sync_copy(ig_hbm.at[pl.ds(base, BPW)], igx_v)
        pltpu.sync_copy(um_hbm.at[pl.ds(base, BPW)], umx_v)
        pltpu.sync_copy(im_hbm.at[pl.ds(base, BPW)], imx_v)

        tabs = [(gu_hbm, ogu_hbm, ugx_v), (gi_hbm, ogi_hbm, igx_v),
                (mu_hbm, omu_hbm, umx_v), (mi_hbm, omi_hbm, imx_v)]
        items = [(t, c) for t in range(4) for c in range(NCHUNK)]
        n = len(items)

        def gather_start(i):
            t, c = items[i]
            tab, _, idx = tabs[t]
            return pltpu.async_copy(
                tab.at[idx.at[pl.ds(c * C, C)]], bufs[i % NBUF],
                gsems[i % NBUF])

        def out_start(i):
            t, c = items[i]
            _, out_r, _ = tabs[t]
            return pltpu.async_copy(
                bufs[i % NBUF], out_r.at[pl.ds(base + c * C, C)],
                osems[i % NBUF])

        gph, oph = {}, {}
        for i in range(min(NBUF, n)):
            gph[i] = gather_start(i)
        for i in range(n):
            gph[i].wait()
            oph[i] = out_start(i)
            if i + NBUF < n:
                oph[i].wait()
                gph[i + NBUF] = gather_start(i + NBUF)
        for i in range(max(0, n - NBUF), n):
            oph[i].wait()

    return k(uidx_g, iidx_g, uidx_m, iidx_m, gu_tab, gi_tab, mu_tab, mi_tab)


def _dense_body(gu_ref, gi_ref, mu_ref, mi_ref, su_ref, si_ref,
                w0u, w0i, b0r, w1, b1r, w2, b2r, wg, wh, boutr, out_ref):
    su = su_ref[...]
    si = si_ref[...]
    gu = jnp.where((su & 1) == 1, gu_ref[:, 64:128], gu_ref[:, 0:64])
    gi = jnp.where((si & 1) == 1, gi_ref[:, 64:128], gi_ref[:, 0:64])

    def sel4(ref, s):
        lo = jnp.where((s & 1) == 1, ref[:, 32:64], ref[:, 0:32])
        hi = jnp.where((s & 1) == 1, ref[:, 96:128], ref[:, 64:96])
        return jnp.where((s & 2) == 2, hi, lo)

    mu = sel4(mu_ref, su)
    mi = sel4(mi_ref, si)
    h = jnp.maximum(mu @ w0u[...] + mi @ w0i[...] + b0r[...], 0.0)
    h = jnp.maximum(h @ w1[...] + b1r[...], 0.0)
    h = jnp.maximum(h @ w2[...] + b2r[...], 0.0)
    g = gu * gi
    p = g @ wg[...] + h @ wh[...] + boutr[...]
    out_ref[...] = jax.nn.sigmoid(p)


def _tc_dense(gu, gi, mu, mi, sel_u, sel_i, W0, b0, W1, b1, W2, b2, Wout, bout):
    BLK = 2048
    grid = (BATCH // BLK,)
    w0u, w0i = W0[:MLP_FACTOR], W0[MLP_FACTOR:]
    wg, wh = Wout[:FACTORS], Wout[FACTORS:]
    b0r = b0.reshape(1, -1)
    b1r = b1.reshape(1, -1)
    b2r = b2.reshape(1, -1)
    boutr = bout.reshape(1, 1)

    def full(a):
        return pl.BlockSpec(a.shape, lambda i: (0,) * a.ndim)

    out = pl.pallas_call(
        _dense_body,
        grid=grid,
        in_specs=[
            pl.BlockSpec((BLK, 128), lambda i: (i, 0)),
            pl.BlockSpec((BLK, 128), lambda i: (i, 0)),
            pl.BlockSpec((BLK, 128), lambda i: (i, 0)),
            pl.BlockSpec((BLK, 128), lambda i: (i, 0)),
            pl.BlockSpec((BLK, 1), lambda i: (i, 0)),
            pl.BlockSpec((BLK, 1), lambda i: (i, 0)),
            full(w0u), full(w0i), full(b0r), full(W1), full(b1r),
            full(W2), full(b2r), full(wg), full(wh), full(boutr),
        ],
        out_specs=pl.BlockSpec((BLK, 1), lambda i: (i, 0)),
        out_shape=jax.ShapeDtypeStruct((BATCH, 1), jnp.float32),
    )(gu, gi, mu, mi, sel_u, sel_i,
      w0u, w0i, b0r, W1, b1r, W2, b2r, wg, wh, boutr)
    return out


def kernel(user_ids, item_ids, gmf_user_emb, gmf_item_emb, mlp_user_emb,
           mlp_item_emb, W0, b0, W1, b1, W2, b2, Wout, bout):
    gu_tab = gmf_user_emb.reshape(-1, 128)
    gi_tab = gmf_item_emb.reshape(-1, 128)
    mu_tab = mlp_user_emb.reshape(-1, 128)
    mi_tab = mlp_item_emb.reshape(-1, 128)
    uidx_g = user_ids >> 1
    iidx_g = item_ids >> 1
    uidx_m = user_ids >> 2
    iidx_m = item_ids >> 2
    gu, gi, mu, mi = _sc_gather(uidx_g, iidx_g, uidx_m, iidx_m,
                                gu_tab, gi_tab, mu_tab, mi_tab)
    sel_u = (user_ids & 3).reshape(BATCH, 1)
    sel_i = (item_ids & 3).reshape(BATCH, 1)
    out = _tc_dense(gu, gi, mu, mi, sel_u, sel_i,
                    W0, b0, W1, b1, W2, b2, Wout, bout)
    return jnp.squeeze(out, axis=-1)


# speedup vs baseline: 1.0187x; 1.0187x over previous
import functools

import jax
import jax.numpy as jnp
from jax import lax
from jax.experimental import pallas as pl
from jax.experimental.pallas import tpu as pltpu
from jax.experimental.pallas import tpu_sc as plsc

BATCH = 16384
FACTORS = 64
MLP_FACTOR = 32
NC, NS = 2, 16
NW = NC * NS
BPW = BATCH // NW

CHUNK = 16
NCHUNKS = BPW // CHUNK
DEPTH = 2


def _sc_gather(user_ids, item_ids, gu_tab, gi_tab, mu_tab, mi_tab):
    mesh = plsc.VectorSubcoreMesh(core_axis_name="c", subcore_axis_name="s")
    out_type = (
        jax.ShapeDtypeStruct((BATCH * FACTORS,), jnp.float32),
        jax.ShapeDtypeStruct((BATCH * FACTORS,), jnp.float32),
        jax.ShapeDtypeStruct((BATCH * MLP_FACTOR,), jnp.float32),
        jax.ShapeDtypeStruct((BATCH * MLP_FACTOR,), jnp.float32),
    )

    @functools.partial(
        pl.kernel,
        mesh=mesh,
        out_type=out_type,
        scratch_types=[
            pltpu.VMEM((BPW,), jnp.int32),
            pltpu.VMEM((BPW,), jnp.int32),
            pltpu.VMEM((BPW * FACTORS,), jnp.float32),
            pltpu.VMEM((BPW * FACTORS,), jnp.float32),
            pltpu.VMEM((BPW * MLP_FACTOR,), jnp.float32),
            pltpu.VMEM((BPW * MLP_FACTOR,), jnp.float32),
            pltpu.SemaphoreType.DMA,
            pltpu.SemaphoreType.DMA,
        ],
    )
    def k(uid_hbm, iid_hbm, gu_hbm, gi_hbm, mu_hbm, mi_hbm,
          ogu_hbm, ogi_hbm, omu_hbm, omi_hbm,
          us_v, is_v, gu_v, gi_v, mu_v, mi_v, gsem, osem):
        wid = lax.axis_index("s") * NC + lax.axis_index("c")
        base = wid * BPW
        pltpu.sync_copy(uid_hbm.at[pl.ds(base, BPW)], us_v)
        pltpu.sync_copy(iid_hbm.at[pl.ds(base, BPW)], is_v)

        def drain_chunk():
            for _ in range(CHUNK):
                pltpu.make_async_copy(gu_hbm.at[pl.ds(0, FACTORS)],
                                      gu_v.at[pl.ds(0, FACTORS)], gsem).wait()
                pltpu.make_async_copy(mu_hbm.at[pl.ds(0, MLP_FACTOR)],
                                      mu_v.at[pl.ds(0, MLP_FACTOR)],
                                      gsem).wait()
                pltpu.make_async_copy(gi_hbm.at[pl.ds(0, FACTORS)],
                                      gi_v.at[pl.ds(0, FACTORS)], gsem).wait()
                pltpu.make_async_copy(mi_hbm.at[pl.ds(0, MLP_FACTOR)],
                                      mi_v.at[pl.ds(0, MLP_FACTOR)],
                                      gsem).wait()

        @pl.loop(0, NCHUNKS)
        def _(c):
            roff = c * CHUNK
            uvec = us_v[pl.ds(roff, CHUNK)]
            ivec = is_v[pl.ds(roff, CHUNK)]
            for j in range(CHUNK):
                i = roff + j
                u = uvec[j]
                it = ivec[j]
                pltpu.async_copy(gu_hbm.at[pl.ds(u * FACTORS, FACTORS)],
                                 gu_v.at[pl.ds(i * FACTORS, FACTORS)], gsem)
                pltpu.async_copy(mu_hbm.at[pl.ds(u * MLP_FACTOR, MLP_FACTOR)],
                                 mu_v.at[pl.ds(i * MLP_FACTOR, MLP_FACTOR)],
                                 gsem)
                pltpu.async_copy(gi_hbm.at[pl.ds(it * FACTORS, FACTORS)],
                                 gi_v.at[pl.ds(i * FACTORS, FACTORS)], gsem)
                pltpu.async_copy(mi_hbm.at[pl.ds(it * MLP_FACTOR, MLP_FACTOR)],
                                 mi_v.at[pl.ds(i * MLP_FACTOR, MLP_FACTOR)],
                                 gsem)

            @pl.when(c >= DEPTH)
            def _():
                drain_chunk()

        for _ in range(min(DEPTH, NCHUNKS)):
            drain_chunk()

        o0 = pltpu.async_copy(
            gu_v, ogu_hbm.at[pl.ds(base * FACTORS, BPW * FACTORS)], osem)
        o1 = pltpu.async_copy(
            gi_v, ogi_hbm.at[pl.ds(base * FACTORS, BPW * FACTORS)], osem)
        o2 = pltpu.async_copy(
            mu_v, omu_hbm.at[pl.ds(base * MLP_FACTOR, BPW * MLP_FACTOR)], osem)
        o3 = pltpu.async_copy(
            mi_v, omi_hbm.at[pl.ds(base * MLP_FACTOR, BPW * MLP_FACTOR)], osem)
        o0.wait()
        o1.wait()
        o2.wait()
        o3.wait()

    return k(user_ids, item_ids, gu_tab, gi_tab, mu_tab, mi_tab)


SBLK = 2048
GROWS = SBLK // 2
MROWS = SBLK // 4


def _dense_body(gu_ref, gi_ref, mu_ref, mi_ref, w0u, w0i, b0r, w1, b1r,
                w2, b2r, whb, wg2, og_ref, om_ref):
    gprod = gu_ref[...] * gi_ref[...]
    og_ref[...] = gprod @ wg2[...]
    h = jnp.maximum(mu_ref[...] @ w0u[...] + mi_ref[...] @ w0i[...] + b0r[...],
                    0.0)
    h = jnp.maximum(h @ w1[...] + b1r[...], 0.0)
    h = jnp.maximum(h @ w2[...] + b2r[...], 0.0)
    om_ref[...] = h @ whb[...]


def _block_diag4(w):
    r, c = w.shape
    z = jnp.zeros((r, c), w.dtype)
    return jnp.block([[w if i == j else z for j in range(4)]
                      for i in range(4)])


def _tc_dense(gu2, gi2, mu2, mi2, W0, b0, W1, b1, W2, b2, Wout, bout):
    grid = (BATCH // SBLK,)
    w0u = _block_diag4(W0[:MLP_FACTOR])
    w0i = _block_diag4(W0[MLP_FACTOR:])
    w1b = _block_diag4(W1)
    w2b = _block_diag4(W2)
    whb = _block_diag4(Wout[FACTORS:])
    wg = Wout[:FACTORS]
    z = jnp.zeros((FACTORS, 1), wg.dtype)
    wg2 = jnp.block([[wg, z], [z, wg]])
    b0r = jnp.tile(b0, 4).reshape(1, -1)
    b1r = jnp.tile(b1, 4).reshape(1, -1)
    b2r = jnp.tile(b2, 4).reshape(1, -1)

    def full(a):
        return pl.BlockSpec(a.shape, lambda i: (0,) * a.ndim)

    og, om = pl.pallas_call(
        _dense_body,
        grid=grid,
        in_specs=[
            pl.BlockSpec((GROWS, 128), lambda i: (i, 0)),
            pl.BlockSpec((GROWS, 128), lambda i: (i, 0)),
            pl.BlockSpec((MROWS, 128), lambda i: (i, 0)),
            pl.BlockSpec((MROWS, 128), lambda i: (i, 0)),
            full(w0u), full(w0i), full(b0r), full(w1b), full(b1r),
            full(w2b), full(b2r), full(whb), full(wg2),
        ],
        out_specs=[
            pl.BlockSpec((GROWS, 2), lambda i: (i, 0)),
            pl.BlockSpec((MROWS, 4), lambda i: (i, 0)),
        ],
        out_shape=[
            jax.ShapeDtypeStruct((BATCH // 2, 2), jnp.float32),
            jax.ShapeDtypeStruct((BATCH // 4, 4), jnp.float32),
        ],
    )(gu2, gi2, mu2, mi2, w0u, w0i, b0r, w1b, b1r, w2b, b2r, whb, wg2)
    return og, om


def kernel(user_ids, item_ids, gmf_user_emb, gmf_item_emb, mlp_user_emb,
           mlp_item_emb, W0, b0, W1, b1, W2, b2, Wout, bout):
    gu1, gi1, mu1, mi1 = _sc_gather(
        user_ids, item_ids, gmf_user_emb.reshape(-1),
        gmf_item_emb.reshape(-1), mlp_user_emb.reshape(-1),
        mlp_item_emb.reshape(-1))
    gu2 = gu1.reshape(BATCH // 2, 128)
    gi2 = gi1.reshape(BATCH // 2, 128)
    mu2 = mu1.reshape(BATCH // 4, 128)
    mi2 = mi1.reshape(BATCH // 4, 128)
    og, om = _tc_dense(gu2, gi2, mu2, mi2,
                       W0, b0, W1, b1, W2, b2, Wout, bout)
    return jax.nn.sigmoid(og.reshape(BATCH) + om.reshape(BATCH) + bout[0])


# speedup vs baseline: 1.4653x; 1.4385x over previous
import functools

import jax
import jax.numpy as jnp
from jax import lax
from jax.experimental import pallas as pl
from jax.experimental.pallas import tpu as pltpu
from jax.experimental.pallas import tpu_sc as plsc

BATCH = 16384
FACTORS = 64
MLP_FACTOR = 32
NC, NS = 2, 16
NW = NC * NS
BPW = BATCH // NW

NPASS = 4
PB = BPW // NPASS
CHUNK = 16
NCHP = PB // CHUNK
DEPTH = 2


def _sc_gather(user_ids, item_ids, gu_tab, gi_tab, mu_tab, mi_tab):
    mesh = plsc.VectorSubcoreMesh(core_axis_name="c", subcore_axis_name="s")
    out_type = (
        jax.ShapeDtypeStruct((BATCH, FACTORS), jnp.float32),
        jax.ShapeDtypeStruct((BATCH, FACTORS), jnp.float32),
        jax.ShapeDtypeStruct((BATCH, MLP_FACTOR), jnp.float32),
        jax.ShapeDtypeStruct((BATCH, MLP_FACTOR), jnp.float32),
    )

    @functools.partial(
        pl.kernel,
        mesh=mesh,
        out_type=out_type,
        scratch_types=[
            pltpu.VMEM((BPW,), jnp.int32),
            pltpu.VMEM((BPW,), jnp.int32),
            pltpu.VMEM((PB, FACTORS), jnp.float32),
            pltpu.VMEM((PB, FACTORS), jnp.float32),
            pltpu.VMEM((PB, MLP_FACTOR), jnp.float32),
            pltpu.VMEM((PB, MLP_FACTOR), jnp.float32),
            pltpu.SemaphoreType.DMA,
            pltpu.SemaphoreType.DMA,
        ],
    )
    def k(uid_hbm, iid_hbm, gu_hbm, gi_hbm, mu_hbm, mi_hbm,
          ogu_hbm, ogi_hbm, omu_hbm, omi_hbm,
          us_v, is_v, gu_v, gi_v, mu_v, mi_v, gsem, osem):
        wid = lax.axis_index("s") * NC + lax.axis_index("c")
        base = wid * BPW
        pltpu.sync_copy(uid_hbm.at[pl.ds(base, BPW)], us_v)
        pltpu.sync_copy(iid_hbm.at[pl.ds(base, BPW)], is_v)

        def drain_chunk():
            for _ in range(CHUNK):
                pltpu.make_async_copy(gu_hbm.at[pl.ds(0, 1), :],
                                      gu_v.at[pl.ds(0, 1), :], gsem).wait()
                pltpu.make_async_copy(mu_hbm.at[pl.ds(0, 1), :],
                                      mu_v.at[pl.ds(0, 1), :], gsem).wait()
                pltpu.make_async_copy(gi_hbm.at[pl.ds(0, 1), :],
                                      gi_v.at[pl.ds(0, 1), :], gsem).wait()
                pltpu.make_async_copy(mi_hbm.at[pl.ds(0, 1), :],
                                      mi_v.at[pl.ds(0, 1), :], gsem).wait()

        for p in range(NPASS):
            poff = p * PB

            @pl.loop(0, NCHP)
            def _(c):
                roff = poff + c * CHUNK
                uvec = us_v[pl.ds(roff, CHUNK)]
                ivec = is_v[pl.ds(roff, CHUNK)]
                for j in range(CHUNK):
                    i = c * CHUNK + j
                    u = uvec[j]
                    it = ivec[j]
                    pltpu.async_copy(gu_hbm.at[pl.ds(u, 1), :],
                                     gu_v.at[pl.ds(i, 1), :], gsem)
                    pltpu.async_copy(mu_hbm.at[pl.ds(u, 1), :],
                                     mu_v.at[pl.ds(i, 1), :], gsem)
                    pltpu.async_copy(gi_hbm.at[pl.ds(it, 1), :],
                                     gi_v.at[pl.ds(i, 1), :], gsem)
                    pltpu.async_copy(mi_hbm.at[pl.ds(it, 1), :],
                                     mi_v.at[pl.ds(i, 1), :], gsem)

                @pl.when(c >= DEPTH)
                def _():
                    drain_chunk()

            for _ in range(min(DEPTH, NCHP)):
                drain_chunk()

            o0 = pltpu.async_copy(
                gu_v, ogu_hbm.at[pl.ds(base + poff, PB), :], osem)
            o1 = pltpu.async_copy(
                gi_v, ogi_hbm.at[pl.ds(base + poff, PB), :], osem)
            o2 = pltpu.async_copy(
                mu_v, omu_hbm.at[pl.ds(base + poff, PB), :], osem)
            o3 = pltpu.async_copy(
                mi_v, omi_hbm.at[pl.ds(base + poff, PB), :], osem)
            o0.wait()
            o1.wait()
            o2.wait()
            o3.wait()

    return k(user_ids, item_ids, gu_tab, gi_tab, mu_tab, mi_tab)


def _dense_body(gu_ref, gi_ref, mu_ref, mi_ref, w0u, w0i, b0r, w1, b1r,
                w2, b2r, wg, wh, boutr, out_ref):
    h = jnp.maximum(mu_ref[...] @ w0u[...] + mi_ref[...] @ w0i[...] + b0r[...],
                    0.0)
    h = jnp.maximum(h @ w1[...] + b1r[...], 0.0)
    h = jnp.maximum(h @ w2[...] + b2r[...], 0.0)
    g = gu_ref[...] * gi_ref[...]
    p = g @ wg[...] + h @ wh[...] + boutr[...]
    out_ref[...] = jax.nn.sigmoid(p)


def _tc_dense(gu, gi, mu, mi, W0, b0, W1, b1, W2, b2, Wout, bout):
    BLK = 2048
    grid = (BATCH // BLK,)
    w0u, w0i = W0[:MLP_FACTOR], W0[MLP_FACTOR:]
    wg, wh = Wout[:FACTORS], Wout[FACTORS:]
    b0r = b0.reshape(1, -1)
    b1r = b1.reshape(1, -1)
    b2r = b2.reshape(1, -1)
    boutr = bout.reshape(1, 1)

    def full(a):
        return pl.BlockSpec(a.shape, lambda i: (0,) * a.ndim)

    out = pl.pallas_call(
        _dense_body,
        grid=grid,
        in_specs=[
            pl.BlockSpec((BLK, FACTORS), lambda i: (i, 0)),
            pl.BlockSpec((BLK, FACTORS), lambda i: (i, 0)),
            pl.BlockSpec((BLK, MLP_FACTOR), lambda i: (i, 0)),
            pl.BlockSpec((BLK, MLP_FACTOR), lambda i: (i, 0)),
            full(w0u), full(w0i), full(b0r), full(W1), full(b1r),
            full(W2), full(b2r), full(wg), full(wh), full(boutr),
        ],
        out_specs=pl.BlockSpec((BLK, 1), lambda i: (i, 0)),
        out_shape=jax.ShapeDtypeStruct((BATCH, 1), jnp.float32),
    )(gu, gi, mu, mi, w0u, w0i, b0r, W1, b1r, W2, b2r, wg, wh, boutr)
    return out


def kernel(user_ids, item_ids, gmf_user_emb, gmf_item_emb, mlp_user_emb,
           mlp_item_emb, W0, b0, W1, b1, W2, b2, Wout, bout):
    gu, gi, mu, mi = _sc_gather(user_ids, item_ids, gmf_user_emb,
                                gmf_item_emb, mlp_user_emb, mlp_item_emb)
    out = _tc_dense(gu, gi, mu, mi, W0, b0, W1, b1, W2, b2, Wout, bout)
    return jnp.squeeze(out, axis=-1)
